# baseline (speedup 1.0000x reference)
import functools

import jax
import jax.numpy as jnp
from jax import lax
from jax.experimental import pallas as pl
from jax.experimental.pallas import tpu as pltpu
from jax.experimental.pallas import tpu_sc as plsc

_N = 10000
_E = 320000
_EMBED = 128
_HID = 256
_NG = 50
_RBF_R = 12.0
_AVG_LEN = 60.0
_CONN = 32.0
_NGRAPH = 64

_NC = 2
_NS = 16
_L = 16
_NW = _NC * _NS
_EPW = _E // _NW

_NSLAB = 5
_ESL = _E // _NSLAB
_EPWS = _ESL // _NW

_GCH = 80
_GITER = _EPWS // _GCH

_SCH = 400
_SITER = _EPW // _SCH

_NP = 10240
_ACC = 4 * _NP

_BE = 1280


_XW = _EMBED // 2


def _gather_body(x_hbm, src_hbm, dst_hbm, xs_out, xd_out,
                 idx_s, idx_d, rows_s, rows_d, sem_s, sem_d):
    wid = lax.axis_index("s") * _NC + lax.axis_index("c")

    def step(c, carry):
        base = wid * _EPWS + c * _GCH
        pltpu.sync_copy(src_hbm.at[pl.ds(base, _GCH)], idx_s)
        pltpu.sync_copy(dst_hbm.at[pl.ds(base, _GCH)], idx_d)
        cp_s = pltpu.async_copy(x_hbm.at[idx_s], rows_s, sem_s)
        cp_d = pltpu.async_copy(x_hbm.at[idx_d], rows_d, sem_d)
        cp_s.wait()
        pltpu.sync_copy(rows_s, xs_out.at[pl.ds(base, _GCH)])
        cp_d.wait()
        pltpu.sync_copy(rows_d, xd_out.at[pl.ds(base, _GCH)])
        return carry

    lax.fori_loop(0, _GITER, step, 0)


def _scatter_body(pairs_hbm, src_hbm, part_out, acc, pairs_v, src_v):
    wid = lax.axis_index("s") * _NC + lax.axis_index("c")
    zero = jnp.zeros((_L,), jnp.float32)

    def zstep(i, carry):
        acc[pl.ds(i * _L, _L)] = zero
        return carry

    lax.fori_loop(0, _ACC // _L, zstep, 0)

    lane = lax.iota(jnp.int32, _L)
    eo = lane >> 2
    fld = lane & 3

    def chunk(c, carry):
        base = wid * _EPW + c * _SCH
        pltpu.sync_copy(src_hbm.at[pl.ds(base, _SCH)], src_v)
        pltpu.sync_copy(pairs_hbm.at[pl.ds(base * 4, _SCH * 4)], pairs_v)

        def grp(j, icarry):
            vals = pairs_v[pl.ds(j * _L, _L)]
            s = plsc.load_gather(src_v, [j * 4 + eo])
            tgt = fld * _NP + s
            for e in range(4):
                plsc.addupdate_scatter(acc, [tgt], vals, mask=eo == e)
            return icarry

        lax.fori_loop(0, _SCH // 4, grp, 0)
        return carry

    lax.fori_loop(0, _SITER, chunk, 0)
    pltpu.sync_copy(acc, part_out.at[wid])


def _mlp_body(xs, xd, dist, vech, wrbf, brbf,
              e_Win, e_bin, e_Wh, e_bh, e_Wout, e_bout,
              f_Win, f_bin, f_Wh, f_bh, f_Wout, f_bout, out_ref):
    step = _RBF_R / (_NG - 1)
    offs = lax.broadcasted_iota(jnp.int32, (1, _NG), 1).astype(jnp.float32) * step
    coeff = -0.5 / step**2
    g = jnp.exp(coeff * (dist[...] - offs) ** 2)
    rbf = jnp.dot(g.astype(jnp.bfloat16), wrbf[...],
                  preferred_element_type=jnp.float32) + brbf[...]
    inp = jnp.concatenate(
        [xs[...].astype(jnp.bfloat16), xd[...].astype(jnp.bfloat16),
         rbf.astype(jnp.bfloat16)], axis=1)

    def silu(v):
        return 0.5 * v * (1.0 + jnp.tanh(0.5 * v))

    def res_mlp(Win, bin_, Wh, bh, Wout, bout):
        h = jnp.dot(inp, Win[...], preferred_element_type=jnp.float32) + bin_[...]
        h = silu(h)
        h2 = jnp.dot(h.astype(jnp.bfloat16), Wh[...],
                     preferred_element_type=jnp.float32) + bh[...]
        h = h + silu(h2)
        return jnp.dot(h.astype(jnp.bfloat16), Wout[...],
                       preferred_element_type=jnp.float32) + bout[...]

    ep = res_mlp(e_Win, e_bin, e_Wh, e_bh, e_Wout, e_bout) * (1.0 / (_AVG_LEN * _CONN))
    fp = res_mlp(f_Win, f_bin, f_Wh, f_bh, f_Wout, f_bout) * (1.0 / _CONN)
    mask0 = (lax.broadcasted_iota(jnp.int32, (1, 4), 1) == 0).astype(jnp.float32)
    out_ref[...] = ep * mask0 + fp * vech[...]


def _finish_body(x_ref, b_ref, energy_ref, ft_ref):
    xv = x_ref[...]
    e_node = jnp.sum(xv[:, 0:_NP], axis=0, keepdims=True)
    ft_ref[0:1, :] = jnp.sum(xv[:, _NP:2 * _NP], axis=0, keepdims=True)
    ft_ref[1:2, :] = jnp.sum(xv[:, 2 * _NP:3 * _NP], axis=0, keepdims=True)
    ft_ref[2:3, :] = jnp.sum(xv[:, 3 * _NP:4 * _NP], axis=0, keepdims=True)
    gid = lax.broadcasted_iota(jnp.int32, (_NGRAPH, 1), 0)
    onehot = (b_ref[...] == gid).astype(jnp.float32)
    energy_ref[...] = lax.dot_general(
        onehot, e_node, (((1,), (1,)), ((), ())),
        preferred_element_type=jnp.float32)


@functools.lru_cache(maxsize=None)
def _sc_calls():
    mesh = plsc.VectorSubcoreMesh(core_axis_name="c", subcore_axis_name="s")
    gather = pl.kernel(
        _gather_body,
        out_type=[jax.ShapeDtypeStruct((_ESL, _EMBED), jnp.float32),
                  jax.ShapeDtypeStruct((_ESL, _EMBED), jnp.float32)],
        mesh=mesh,
        scratch_types=[
            pltpu.VMEM((_GCH,), jnp.int32),
            pltpu.VMEM((_GCH,), jnp.int32),
            pltpu.VMEM((_GCH, _EMBED), jnp.float32),
            pltpu.VMEM((_GCH, _EMBED), jnp.float32),
            pltpu.SemaphoreType.DMA,
            pltpu.SemaphoreType.DMA,
        ],
    )
    scatter = pl.kernel(
        _scatter_body,
        out_type=[jax.ShapeDtypeStruct((_NW, _ACC), jnp.float32)],
        mesh=mesh,
        compiler_params=pltpu.CompilerParams(needs_layout_passes=False),
        scratch_types=[
            pltpu.VMEM((_ACC,), jnp.float32),
            pltpu.VMEM((_SCH * 4,), jnp.float32),
            pltpu.VMEM((_SCH,), jnp.int32),
        ],
    )
    return gather, scatter


def _mlp_call(xs, xd, dist2, vech4, *weights):
    grid = (_ESL // _BE,)
    edge_spec = lambda width: pl.BlockSpec((_BE, width), lambda i: (i, 0))
    w_spec = lambda a, b: pl.BlockSpec((a, b), lambda i: (0, 0))
    in_specs = [
        edge_spec(_EMBED), edge_spec(_EMBED), edge_spec(1), edge_spec(4),
        w_spec(_NG, _EMBED), w_spec(1, _EMBED),
        w_spec(3 * _EMBED, _HID), w_spec(1, _HID),
        w_spec(_HID, _HID), w_spec(1, _HID),
        w_spec(_HID, 1), w_spec(1, 1),
        w_spec(3 * _EMBED, _HID), w_spec(1, _HID),
        w_spec(_HID, _HID), w_spec(1, _HID),
        w_spec(_HID, 1), w_spec(1, 1),
    ]
    return pl.pallas_call(
        _mlp_body,
        grid=grid,
        in_specs=in_specs,
        out_specs=pl.BlockSpec((_BE, 4), lambda i: (i, 0)),
        out_shape=jax.ShapeDtypeStruct((_ESL, 4), jnp.float32),
    )(xs, xd, dist2, vech4, *weights)


def _finish_call(x, batch2):
    return pl.pallas_call(
        _finish_body,
        in_specs=[pl.BlockSpec((_NW, _ACC), lambda: (0, 0)),
                  pl.BlockSpec((1, _NP), lambda: (0, 0))],
        out_specs=[pl.BlockSpec((_NGRAPH, 1), lambda: (0, 0)),
                   pl.BlockSpec((3, _NP), lambda: (0, 0))],
        out_shape=[jax.ShapeDtypeStruct((_NGRAPH, 1), jnp.float32),
                   jax.ShapeDtypeStruct((3, _NP), jnp.float32)],
    )(x, batch2)


def kernel(x, edge_index, batch, dist, vec_hat,
           W_rbf, b_rbf,
           e_Win, e_bin, e_Wh, e_bh, e_Wout, e_bout,
           f_Win, f_bin, f_Wh, f_bh, f_Wout, f_bout):
    gather_call, scatter_call = _sc_calls()
    src = edge_index[0]
    dst = edge_index[1]
    dist2 = dist.reshape(_E, 1)
    vech4 = jnp.concatenate(
        [jnp.zeros((_E, 1), jnp.float32), vec_hat], axis=1)
    bf = jnp.bfloat16
    weights = (
        W_rbf.astype(bf), b_rbf.reshape(1, _EMBED),
        e_Win.astype(bf), e_bin.reshape(1, _HID),
        e_Wh.astype(bf), e_bh.reshape(1, _HID),
        e_Wout.astype(bf), e_bout.reshape(1, 1),
        f_Win.astype(bf), f_bin.reshape(1, _HID),
        f_Wh.astype(bf), f_bh.reshape(1, _HID),
        f_Wout.astype(bf), f_bout.reshape(1, 1))
    pairs_slabs = []
    for k in range(_NSLAB):
        sl = slice(k * _ESL, (k + 1) * _ESL)
        xs, xd = gather_call(x, src[sl], dst[sl])
        pairs_slabs.append(
            _mlp_call(xs, xd, dist2[sl], vech4[sl], *weights))
    pairs = jnp.concatenate(pairs_slabs, axis=0)
    (partials,) = scatter_call(pairs.reshape(_E * 4), src)
    batch_p = jnp.full((1, _NP), _NGRAPH, jnp.int32).at[0, :_N].set(batch)
    energy, ft = _finish_call(partials, batch_p)
    forces = ft[:, :_N].T
    return (energy, forces)


# speedup vs baseline: 1.1381x; 1.1381x over previous
import functools

import jax
import jax.numpy as jnp
from jax import lax
from jax.experimental import pallas as pl
from jax.experimental.pallas import tpu as pltpu
from jax.experimental.pallas import tpu_sc as plsc

_N = 10000
_E = 320000
_EMBED = 128
_HID = 256
_NG = 50
_RBF_R = 12.0
_AVG_LEN = 60.0
_CONN = 32.0
_NGRAPH = 64

_NC = 2
_NS = 16
_L = 16
_NW = _NC * _NS
_EPW = _E // _NW

_GCH = 80
_GITER = _EPW // _GCH

_SCH = 400
_SITER = _EPW // _SCH

_NP = 10240
_ACC = 4 * _NP

_BE = 1280


_XW = _EMBED // 2


def _gather_body(x_hbm, src_hbm, dst_hbm, xs_out, xd_out,
                 idx_s, idx_d, rows_s, rows_d, sem_s, sem_d):
    wid = lax.axis_index("s") * _NC + lax.axis_index("c")

    def step(c, carry):
        base = wid * _EPW + c * _GCH
        pltpu.sync_copy(src_hbm.at[pl.ds(base, _GCH)], idx_s)
        pltpu.sync_copy(dst_hbm.at[pl.ds(base, _GCH)], idx_d)
        cp_s = pltpu.async_copy(x_hbm.at[idx_s], rows_s, sem_s)
        cp_d = pltpu.async_copy(x_hbm.at[idx_d], rows_d, sem_d)
        cp_s.wait()
        pltpu.sync_copy(rows_s, xs_out.at[pl.ds(base, _GCH)])
        cp_d.wait()
        pltpu.sync_copy(rows_d, xd_out.at[pl.ds(base, _GCH)])
        return carry

    lax.fori_loop(0, _GITER, step, 0)


def _scatter_body(pairs_hbm, src_hbm, part_out, acc, pairs_v, src_v):
    wid = lax.axis_index("s") * _NC + lax.axis_index("c")
    zero = jnp.zeros((_L,), jnp.float32)

    def zstep(i, carry):
        acc[pl.ds(i * _L, _L)] = zero
        return carry

    lax.fori_loop(0, _ACC // _L, zstep, 0)

    lane = lax.iota(jnp.int32, _L)
    eo = lane >> 2
    fld = lane & 3

    def chunk(c, carry):
        base = wid * _EPW + c * _SCH
        pltpu.sync_copy(src_hbm.at[pl.ds(base, _SCH)], src_v)
        pltpu.sync_copy(pairs_hbm.at[pl.ds(base * 4, _SCH * 4)], pairs_v)

        def grp(j, icarry):
            vals = pairs_v[pl.ds(j * _L, _L)]
            s = plsc.load_gather(src_v, [j * 4 + eo])
            tgt = fld * _NP + s
            for e in range(4):
                plsc.addupdate_scatter(acc, [tgt], vals, mask=eo == e)
            return icarry

        lax.fori_loop(0, _SCH // 4, grp, 0)
        return carry

    lax.fori_loop(0, _SITER, chunk, 0)
    pltpu.sync_copy(acc, part_out.at[wid])


def _mlp_body(xs, xd, dist, vech, wrbf, brbf,
              e_Win, e_bin, e_Wh, e_bh, e_Wout, e_bout,
              f_Win, f_bin, f_Wh, f_bh, f_Wout, f_bout, out_ref):
    step = _RBF_R / (_NG - 1)
    offs = lax.broadcasted_iota(jnp.int32, (1, _NG), 1).astype(jnp.float32) * step
    coeff = -0.5 / step**2
    g = jnp.exp(coeff * (dist[...] - offs) ** 2)
    rbf = jnp.dot(g.astype(jnp.bfloat16), wrbf[...],
                  preferred_element_type=jnp.float32) + brbf[...]
    inp = jnp.concatenate(
        [xs[...].astype(jnp.bfloat16), xd[...].astype(jnp.bfloat16),
         rbf.astype(jnp.bfloat16)], axis=1)

    def silu(v):
        return 0.5 * v * (1.0 + jnp.tanh(0.5 * v))

    def res_mlp(Win, bin_, Wh, bh, Wout, bout):
        h = jnp.dot(inp, Win[...], preferred_element_type=jnp.float32) + bin_[...]
        h = silu(h)
        h2 = jnp.dot(h.astype(jnp.bfloat16), Wh[...],
                     preferred_element_type=jnp.float32) + bh[...]
        h = h + silu(h2)
        return jnp.dot(h.astype(jnp.bfloat16), Wout[...],
                       preferred_element_type=jnp.float32) + bout[...]

    ep = res_mlp(e_Win, e_bin, e_Wh, e_bh, e_Wout, e_bout) * (1.0 / (_AVG_LEN * _CONN))
    fp = res_mlp(f_Win, f_bin, f_Wh, f_bh, f_Wout, f_bout) * (1.0 / _CONN)
    out_ref[...] = jnp.concatenate([ep, fp * vech[...]], axis=1)


def _finish_body(x_ref, b_ref, energy_ref, ft_ref):
    xv = x_ref[...]
    e_node = jnp.sum(xv[:, 0:_NP], axis=0, keepdims=True)
    ft_ref[0:1, :] = jnp.sum(xv[:, _NP:2 * _NP], axis=0, keepdims=True)
    ft_ref[1:2, :] = jnp.sum(xv[:, 2 * _NP:3 * _NP], axis=0, keepdims=True)
    ft_ref[2:3, :] = jnp.sum(xv[:, 3 * _NP:4 * _NP], axis=0, keepdims=True)
    gid = lax.broadcasted_iota(jnp.int32, (_NGRAPH, 1), 0)
    onehot = (b_ref[...] == gid).astype(jnp.float32)
    energy_ref[...] = lax.dot_general(
        onehot, e_node, (((1,), (1,)), ((), ())),
        preferred_element_type=jnp.float32)


@functools.lru_cache(maxsize=None)
def _sc_calls():
    mesh = plsc.VectorSubcoreMesh(core_axis_name="c", subcore_axis_name="s")
    gather = pl.kernel(
        _gather_body,
        out_type=[jax.ShapeDtypeStruct((_E, _EMBED), jnp.float32),
                  jax.ShapeDtypeStruct((_E, _EMBED), jnp.float32)],
        mesh=mesh,
        scratch_types=[
            pltpu.VMEM((_GCH,), jnp.int32),
            pltpu.VMEM((_GCH,), jnp.int32),
            pltpu.VMEM((_GCH, _EMBED), jnp.float32),
            pltpu.VMEM((_GCH, _EMBED), jnp.float32),
            pltpu.SemaphoreType.DMA,
            pltpu.SemaphoreType.DMA,
        ],
    )
    scatter = pl.kernel(
        _scatter_body,
        out_type=[jax.ShapeDtypeStruct((_NW, _ACC), jnp.float32)],
        mesh=mesh,
        compiler_params=pltpu.CompilerParams(needs_layout_passes=False),
        scratch_types=[
            pltpu.VMEM((_ACC,), jnp.float32),
            pltpu.VMEM((_SCH * 4,), jnp.float32),
            pltpu.VMEM((_SCH,), jnp.int32),
        ],
    )
    return gather, scatter


def _mlp_call(xs, xd, dist2, vech3, *weights):
    grid = (_E // _BE,)
    edge_spec = lambda width: pl.BlockSpec((_BE, width), lambda i: (i, 0))
    w_spec = lambda a, b: pl.BlockSpec((a, b), lambda i: (0, 0))
    in_specs = [
        edge_spec(_EMBED), edge_spec(_EMBED), edge_spec(1), edge_spec(3),
        w_spec(_NG, _EMBED), w_spec(1, _EMBED),
        w_spec(3 * _EMBED, _HID), w_spec(1, _HID),
        w_spec(_HID, _HID), w_spec(1, _HID),
        w_spec(_HID, 1), w_spec(1, 1),
        w_spec(3 * _EMBED, _HID), w_spec(1, _HID),
        w_spec(_HID, _HID), w_spec(1, _HID),
        w_spec(_HID, 1), w_spec(1, 1),
    ]
    return pl.pallas_call(
        _mlp_body,
        grid=grid,
        in_specs=in_specs,
        out_specs=pl.BlockSpec((_BE, 4), lambda i: (i, 0)),
        out_shape=jax.ShapeDtypeStruct((_E, 4), jnp.float32),
    )(xs, xd, dist2, vech3, *weights)


def _finish_call(x, batch2):
    return pl.pallas_call(
        _finish_body,
        in_specs=[pl.BlockSpec((_NW, _ACC), lambda: (0, 0)),
                  pl.BlockSpec((1, _NP), lambda: (0, 0))],
        out_specs=[pl.BlockSpec((_NGRAPH, 1), lambda: (0, 0)),
                   pl.BlockSpec((3, _NP), lambda: (0, 0))],
        out_shape=[jax.ShapeDtypeStruct((_NGRAPH, 1), jnp.float32),
                   jax.ShapeDtypeStruct((3, _NP), jnp.float32)],
    )(x, batch2)


def kernel(x, edge_index, batch, dist, vec_hat,
           W_rbf, b_rbf,
           e_Win, e_bin, e_Wh, e_bh, e_Wout, e_bout,
           f_Win, f_bin, f_Wh, f_bh, f_Wout, f_bout):
    gather_call, scatter_call = _sc_calls()
    src = edge_index[0]
    dst = edge_index[1]
    dist2 = dist.reshape(_E, 1)
    bf = jnp.bfloat16
    weights = (
        W_rbf.astype(bf), b_rbf.reshape(1, _EMBED),
        e_Win.astype(bf), e_bin.reshape(1, _HID),
        e_Wh.astype(bf), e_bh.reshape(1, _HID),
        e_Wout.astype(bf), e_bout.reshape(1, 1),
        f_Win.astype(bf), f_bin.reshape(1, _HID),
        f_Wh.astype(bf), f_bh.reshape(1, _HID),
        f_Wout.astype(bf), f_bout.reshape(1, 1))
    xs, xd = gather_call(x, src, dst)
    pairs = _mlp_call(xs, xd, dist2, vec_hat, *weights)
    (partials,) = scatter_call(pairs.reshape(_E * 4), src)
    batch_p = jnp.full((1, _NP), _NGRAPH, jnp.int32).at[0, :_N].set(batch)
    energy, ft = _finish_call(partials, batch_p)
    forces = ft[:, :_N].T
    return (energy, forces)


# speedup vs baseline: 1.2267x; 1.0778x over previous
import functools

import jax
import jax.numpy as jnp
from jax import lax
from jax.experimental import pallas as pl
from jax.experimental.pallas import tpu as pltpu
from jax.experimental.pallas import tpu_sc as plsc

_N = 10000
_E = 320000
_EMBED = 128
_HID = 256
_NG = 50
_RBF_R = 12.0
_AVG_LEN = 60.0
_CONN = 32.0
_NGRAPH = 64

_NC = 2
_NS = 16
_L = 16
_NW = _NC * _NS
_EPW = _E // _NW

_GCH = 80
_GITER = _EPW // _GCH

_SCH = 400
_SITER = _EPW // _SCH

_NP = 10240
_ACC = 4 * _NP

_BE = 1280


_XW = _EMBED // 2


def _gather_body(x_hbm, src_hbm, dst_hbm, xs_out, xd_out,
                 idx_s0, idx_s1, idx_d0, idx_d1,
                 rows_s0, rows_s1, rows_d0, rows_d1,
                 gs0, gs1, gd0, gd1, ws0, ws1, wd0, wd1):
    wid = lax.axis_index("s") * _NC + lax.axis_index("c")
    base0 = wid * _EPW
    idx_s = (idx_s0, idx_s1)
    idx_d = (idx_d0, idx_d1)
    rows_s = (rows_s0, rows_s1)
    rows_d = (rows_d0, rows_d1)
    gsem_s = (gs0, gs1)
    gsem_d = (gd0, gd1)
    wsem_s = (ws0, ws1)
    wsem_d = (wd0, wd1)

    def load_idx(i, par):
        pltpu.sync_copy(src_hbm.at[pl.ds(base0 + i * _GCH, _GCH)], idx_s[par])
        pltpu.sync_copy(dst_hbm.at[pl.ds(base0 + i * _GCH, _GCH)], idx_d[par])

    def start_gather(par):
        pltpu.async_copy(x_hbm.at[idx_s[par]], rows_s[par], gsem_s[par])
        pltpu.async_copy(x_hbm.at[idx_d[par]], rows_d[par], gsem_d[par])

    def wait_gather(par):
        pltpu.make_async_copy(x_hbm.at[idx_s[par]], rows_s[par], gsem_s[par]).wait()
        pltpu.make_async_copy(x_hbm.at[idx_d[par]], rows_d[par], gsem_d[par]).wait()

    def start_writeout(i, par):
        dst = pl.ds(base0 + i * _GCH, _GCH)
        pltpu.async_copy(rows_s[par], xs_out.at[dst], wsem_s[par])
        pltpu.async_copy(rows_d[par], xd_out.at[dst], wsem_d[par])

    def wait_writeout(par):
        dst = pl.ds(0, _GCH)
        pltpu.make_async_copy(rows_s[par], xs_out.at[dst], wsem_s[par]).wait()
        pltpu.make_async_copy(rows_d[par], xd_out.at[dst], wsem_d[par]).wait()

    def advance(i, par):
        nxt = 1 - par

        @pl.when(i + 1 < _GITER)
        def _():
            load_idx(i + 1, nxt)

            @pl.when(i >= 1)
            def _():
                wait_writeout(nxt)

            start_gather(nxt)

        wait_gather(par)
        start_writeout(i, par)

    load_idx(0, 0)
    start_gather(0)

    def step(g, carry):
        advance(2 * g, 0)
        advance(2 * g + 1, 1)
        return carry

    lax.fori_loop(0, _GITER // 2, step, 0)
    wait_gather(0)
    start_writeout(_GITER - 1, 0)
    wait_writeout(1)
    wait_writeout(0)


def _scatter_body(pairs_hbm, src_hbm, part_out, acc, pairs_v, src_v):
    wid = lax.axis_index("s") * _NC + lax.axis_index("c")
    zero = jnp.zeros((_L,), jnp.float32)

    def zstep(i, carry):
        acc[pl.ds(i * _L, _L)] = zero
        return carry

    lax.fori_loop(0, _ACC // _L, zstep, 0)

    lane = lax.iota(jnp.int32, _L)
    eo = lane >> 2
    fld = lane & 3

    def chunk(c, carry):
        base = wid * _EPW + c * _SCH
        pltpu.sync_copy(src_hbm.at[pl.ds(base, _SCH)], src_v)
        pltpu.sync_copy(pairs_hbm.at[pl.ds(base * 4, _SCH * 4)], pairs_v)

        def grp(j, icarry):
            vals = pairs_v[pl.ds(j * _L, _L)]
            s = plsc.load_gather(src_v, [j * 4 + eo])
            tgt = fld * _NP + s
            for e in range(4):
                plsc.addupdate_scatter(acc, [tgt], vals, mask=eo == e)
            return icarry

        lax.fori_loop(0, _SCH // 4, grp, 0)
        return carry

    lax.fori_loop(0, _SITER, chunk, 0)
    pltpu.sync_copy(acc, part_out.at[wid])


def _mlp_body(xs, xd, dist, vech, wrbf, brbf,
              e_Win, e_bin, e_Wh, e_bh, e_Wout, e_bout,
              f_Win, f_bin, f_Wh, f_bh, f_Wout, f_bout, out_ref):
    step = _RBF_R / (_NG - 1)
    offs = lax.broadcasted_iota(jnp.int32, (1, _NG), 1).astype(jnp.float32) * step
    coeff = -0.5 / step**2
    g = jnp.exp(coeff * (dist[...] - offs) ** 2)
    rbf = jnp.dot(g.astype(jnp.bfloat16), wrbf[...],
                  preferred_element_type=jnp.float32) + brbf[...]
    inp = jnp.concatenate(
        [xs[...].astype(jnp.bfloat16), xd[...].astype(jnp.bfloat16),
         rbf.astype(jnp.bfloat16)], axis=1)

    def silu(v):
        return 0.5 * v * (1.0 + jnp.tanh(0.5 * v))

    def res_mlp(Win, bin_, Wh, bh, Wout, bout):
        h = jnp.dot(inp, Win[...], preferred_element_type=jnp.float32) + bin_[...]
        h = silu(h)
        h2 = jnp.dot(h.astype(jnp.bfloat16), Wh[...],
                     preferred_element_type=jnp.float32) + bh[...]
        h = h + silu(h2)
        return jnp.dot(h.astype(jnp.bfloat16), Wout[...],
                       preferred_element_type=jnp.float32) + bout[...]

    ep = res_mlp(e_Win, e_bin, e_Wh, e_bh, e_Wout, e_bout) * (1.0 / (_AVG_LEN * _CONN))
    fp = res_mlp(f_Win, f_bin, f_Wh, f_bh, f_Wout, f_bout) * (1.0 / _CONN)
    out_ref[...] = jnp.concatenate([ep, fp * vech[...]], axis=1)


def _finish_body(x_ref, b_ref, energy_ref, ft_ref):
    xv = x_ref[...]
    e_node = jnp.sum(xv[:, 0:_NP], axis=0, keepdims=True)
    ft_ref[0:1, :] = jnp.sum(xv[:, _NP:2 * _NP], axis=0, keepdims=True)
    ft_ref[1:2, :] = jnp.sum(xv[:, 2 * _NP:3 * _NP], axis=0, keepdims=True)
    ft_ref[2:3, :] = jnp.sum(xv[:, 3 * _NP:4 * _NP], axis=0, keepdims=True)
    gid = lax.broadcasted_iota(jnp.int32, (_NGRAPH, 1), 0)
    onehot = (b_ref[...] == gid).astype(jnp.float32)
    energy_ref[...] = lax.dot_general(
        onehot, e_node, (((1,), (1,)), ((), ())),
        preferred_element_type=jnp.float32)


@functools.lru_cache(maxsize=None)
def _sc_calls():
    mesh = plsc.VectorSubcoreMesh(core_axis_name="c", subcore_axis_name="s")
    gather = pl.kernel(
        _gather_body,
        out_type=[jax.ShapeDtypeStruct((_E, _EMBED), jnp.float32),
                  jax.ShapeDtypeStruct((_E, _EMBED), jnp.float32)],
        mesh=mesh,
        scratch_types=(
            [pltpu.VMEM((_GCH,), jnp.int32)] * 4
            + [pltpu.VMEM((_GCH, _EMBED), jnp.float32)] * 4
            + [pltpu.SemaphoreType.DMA] * 8
        ),
    )
    scatter = pl.kernel(
        _scatter_body,
        out_type=[jax.ShapeDtypeStruct((_NW, _ACC), jnp.float32)],
        mesh=mesh,
        compiler_params=pltpu.CompilerParams(needs_layout_passes=False),
        scratch_types=[
            pltpu.VMEM((_ACC,), jnp.float32),
            pltpu.VMEM((_SCH * 4,), jnp.float32),
            pltpu.VMEM((_SCH,), jnp.int32),
        ],
    )
    return gather, scatter


def _mlp_call(xs, xd, dist2, vech3, *weights):
    grid = (_E // _BE,)
    edge_spec = lambda width: pl.BlockSpec((_BE, width), lambda i: (i, 0))
    w_spec = lambda a, b: pl.BlockSpec((a, b), lambda i: (0, 0))
    in_specs = [
        edge_spec(_EMBED), edge_spec(_EMBED), edge_spec(1), edge_spec(3),
        w_spec(_NG, _EMBED), w_spec(1, _EMBED),
        w_spec(3 * _EMBED, _HID), w_spec(1, _HID),
        w_spec(_HID, _HID), w_spec(1, _HID),
        w_spec(_HID, 1), w_spec(1, 1),
        w_spec(3 * _EMBED, _HID), w_spec(1, _HID),
        w_spec(_HID, _HID), w_spec(1, _HID),
        w_spec(_HID, 1), w_spec(1, 1),
    ]
    return pl.pallas_call(
        _mlp_body,
        grid=grid,
        in_specs=in_specs,
        out_specs=pl.BlockSpec((_BE, 4), lambda i: (i, 0)),
        out_shape=jax.ShapeDtypeStruct((_E, 4), jnp.float32),
    )(xs, xd, dist2, vech3, *weights)


def _finish_call(x, batch2):
    return pl.pallas_call(
        _finish_body,
        in_specs=[pl.BlockSpec((_NW, _ACC), lambda: (0, 0)),
                  pl.BlockSpec((1, _NP), lambda: (0, 0))],
        out_specs=[pl.BlockSpec((_NGRAPH, 1), lambda: (0, 0)),
                   pl.BlockSpec((3, _NP), lambda: (0, 0))],
        out_shape=[jax.ShapeDtypeStruct((_NGRAPH, 1), jnp.float32),
                   jax.ShapeDtypeStruct((3, _NP), jnp.float32)],
    )(x, batch2)


def kernel(x, edge_index, batch, dist, vec_hat,
           W_rbf, b_rbf,
           e_Win, e_bin, e_Wh, e_bh, e_Wout, e_bout,
           f_Win, f_bin, f_Wh, f_bh, f_Wout, f_bout):
    gather_call, scatter_call = _sc_calls()
    src = edge_index[0]
    dst = edge_index[1]
    dist2 = dist.reshape(_E, 1)
    bf = jnp.bfloat16
    weights = (
        W_rbf.astype(bf), b_rbf.reshape(1, _EMBED),
        e_Win.astype(bf), e_bin.reshape(1, _HID),
        e_Wh.astype(bf), e_bh.reshape(1, _HID),
        e_Wout.astype(bf), e_bout.reshape(1, 1),
        f_Win.astype(bf), f_bin.reshape(1, _HID),
        f_Wh.astype(bf), f_bh.reshape(1, _HID),
        f_Wout.astype(bf), f_bout.reshape(1, 1))
    xs, xd = gather_call(x, src, dst)
    pairs = _mlp_call(xs, xd, dist2, vec_hat, *weights)
    (partials,) = scatter_call(pairs.reshape(_E * 4), src)
    batch_p = jnp.full((1, _NP), _NGRAPH, jnp.int32).at[0, :_N].set(batch)
    energy, ft = _finish_call(partials, batch_p)
    forces = ft[:, :_N].T
    return (energy, forces)


# speedup vs baseline: 1.2898x; 1.0514x over previous
import functools

import jax
import jax.numpy as jnp
from jax import lax
from jax.experimental import pallas as pl
from jax.experimental.pallas import tpu as pltpu
from jax.experimental.pallas import tpu_sc as plsc

_N = 10000
_E = 320000
_EMBED = 128
_HID = 256
_NG = 50
_RBF_R = 12.0
_AVG_LEN = 60.0
_CONN = 32.0
_NGRAPH = 64

_NC = 2
_NS = 16
_L = 16
_NW = _NC * _NS
_EPW = _E // _NW

_GCH = 80
_GITER = _EPW // _GCH

_SCH = 400
_SITER = _EPW // _SCH

_NP = 10240
_ACC = 4 * _NP

_BE = 2560


_XW = _EMBED // 2


def _gather_body(x_hbm, src_hbm, dst_hbm, xs_out, xd_out,
                 idx_s0, idx_s1, idx_d0, idx_d1,
                 rows_s0, rows_s1, rows_d0, rows_d1,
                 gs0, gs1, gd0, gd1, ws0, ws1, wd0, wd1):
    wid = lax.axis_index("s") * _NC + lax.axis_index("c")
    base0 = wid * _EPW
    idx_s = (idx_s0, idx_s1)
    idx_d = (idx_d0, idx_d1)
    rows_s = (rows_s0, rows_s1)
    rows_d = (rows_d0, rows_d1)
    gsem_s = (gs0, gs1)
    gsem_d = (gd0, gd1)
    wsem_s = (ws0, ws1)
    wsem_d = (wd0, wd1)

    def load_idx(i, par):
        pltpu.sync_copy(src_hbm.at[pl.ds(base0 + i * _GCH, _GCH)], idx_s[par])
        pltpu.sync_copy(dst_hbm.at[pl.ds(base0 + i * _GCH, _GCH)], idx_d[par])

    def start_gather(par):
        pltpu.async_copy(x_hbm.at[idx_s[par]], rows_s[par], gsem_s[par])
        pltpu.async_copy(x_hbm.at[idx_d[par]], rows_d[par], gsem_d[par])

    def wait_gather(par):
        pltpu.make_async_copy(x_hbm.at[idx_s[par]], rows_s[par], gsem_s[par]).wait()
        pltpu.make_async_copy(x_hbm.at[idx_d[par]], rows_d[par], gsem_d[par]).wait()

    def start_writeout(i, par):
        dst = pl.ds(base0 + i * _GCH, _GCH)
        pltpu.async_copy(rows_s[par], xs_out.at[dst], wsem_s[par])
        pltpu.async_copy(rows_d[par], xd_out.at[dst], wsem_d[par])

    def wait_writeout(par):
        dst = pl.ds(0, _GCH)
        pltpu.make_async_copy(rows_s[par], xs_out.at[dst], wsem_s[par]).wait()
        pltpu.make_async_copy(rows_d[par], xd_out.at[dst], wsem_d[par]).wait()

    def advance(i, par):
        nxt = 1 - par

        @pl.when(i + 1 < _GITER)
        def _():
            load_idx(i + 1, nxt)

            @pl.when(i >= 1)
            def _():
                wait_writeout(nxt)

            start_gather(nxt)

        wait_gather(par)
        start_writeout(i, par)

    load_idx(0, 0)
    start_gather(0)

    def step(g, carry):
        advance(2 * g, 0)
        advance(2 * g + 1, 1)
        return carry

    lax.fori_loop(0, _GITER // 2, step, 0)
    wait_gather(0)
    start_writeout(_GITER - 1, 0)
    wait_writeout(1)
    wait_writeout(0)


def _scatter_body(pairs_hbm, src_hbm, part_out, acc, pairs_v, src_v):
    wid = lax.axis_index("s") * _NC + lax.axis_index("c")
    zero = jnp.zeros((_L,), jnp.float32)

    def zstep(i, carry):
        acc[pl.ds(i * _L, _L)] = zero
        return carry

    lax.fori_loop(0, _ACC // _L, zstep, 0)

    lane = lax.iota(jnp.int32, _L)
    eo = lane >> 2
    fld = lane & 3

    def chunk(c, carry):
        base = wid * _EPW + c * _SCH
        pltpu.sync_copy(src_hbm.at[pl.ds(base, _SCH)], src_v)
        pltpu.sync_copy(pairs_hbm.at[pl.ds(base * 4, _SCH * 4)], pairs_v)

        def grp(j, icarry):
            vals = pairs_v[pl.ds(j * _L, _L)]
            s = plsc.load_gather(src_v, [j * 4 + eo])
            tgt = fld * _NP + s
            for e in range(4):
                plsc.addupdate_scatter(acc, [tgt], vals, mask=eo == e)
            return icarry

        lax.fori_loop(0, _SCH // 4, grp, 0)
        return carry

    lax.fori_loop(0, _SITER, chunk, 0)
    pltpu.sync_copy(acc, part_out.at[wid])


def _mlp_body(xs, xd, dist, vech, W1, W2, Wg, b1,
              e_Wh, e_bh, e_Wout, e_bout,
              f_Wh, f_bh, f_Wout, f_bout, out_ref):
    step = _RBF_R / (_NG - 1)
    offs = lax.broadcasted_iota(jnp.int32, (1, _NG), 1).astype(jnp.float32) * step
    coeff = -0.5 / step**2
    g = jnp.exp(coeff * (dist[...] - offs) ** 2)

    def silu(v):
        return 0.5 * v * (1.0 + jnp.tanh(0.5 * v))

    h = (jnp.dot(xs[...].astype(jnp.bfloat16), W1[...],
                 preferred_element_type=jnp.float32)
         + jnp.dot(xd[...].astype(jnp.bfloat16), W2[...],
                   preferred_element_type=jnp.float32)
         + jnp.dot(g.astype(jnp.bfloat16), Wg[...],
                   preferred_element_type=jnp.float32)
         + b1[...])
    h = silu(h)

    def head(h1, Wh, bh, Wout, bout):
        h2 = jnp.dot(h1.astype(jnp.bfloat16), Wh[...],
                     preferred_element_type=jnp.float32) + bh[...]
        h2 = h1 + silu(h2)
        return jnp.dot(h2.astype(jnp.bfloat16), Wout[...],
                       preferred_element_type=jnp.float32) + bout[...]

    ep = head(h[:, :_HID], e_Wh, e_bh, e_Wout, e_bout) * (1.0 / (_AVG_LEN * _CONN))
    fp = head(h[:, _HID:], f_Wh, f_bh, f_Wout, f_bout) * (1.0 / _CONN)
    out_ref[...] = jnp.concatenate([ep, fp * vech[...]], axis=1)


def _finish_body(x_ref, b_ref, energy_ref, ft_ref):
    xv = x_ref[...]
    e_node = jnp.sum(xv[:, 0:_NP], axis=0, keepdims=True)
    ft_ref[0:1, :] = jnp.sum(xv[:, _NP:2 * _NP], axis=0, keepdims=True)
    ft_ref[1:2, :] = jnp.sum(xv[:, 2 * _NP:3 * _NP], axis=0, keepdims=True)
    ft_ref[2:3, :] = jnp.sum(xv[:, 3 * _NP:4 * _NP], axis=0, keepdims=True)
    gid = lax.broadcasted_iota(jnp.int32, (_NGRAPH, 1), 0)
    onehot = (b_ref[...] == gid).astype(jnp.float32)
    energy_ref[...] = lax.dot_general(
        onehot, e_node, (((1,), (1,)), ((), ())),
        preferred_element_type=jnp.float32)


@functools.lru_cache(maxsize=None)
def _sc_calls():
    mesh = plsc.VectorSubcoreMesh(core_axis_name="c", subcore_axis_name="s")
    gather = pl.kernel(
        _gather_body,
        out_type=[jax.ShapeDtypeStruct((_E, _EMBED), jnp.float32),
                  jax.ShapeDtypeStruct((_E, _EMBED), jnp.float32)],
        mesh=mesh,
        scratch_types=(
            [pltpu.VMEM((_GCH,), jnp.int32)] * 4
            + [pltpu.VMEM((_GCH, _EMBED), jnp.float32)] * 4
            + [pltpu.SemaphoreType.DMA] * 8
        ),
    )
    scatter = pl.kernel(
        _scatter_body,
        out_type=[jax.ShapeDtypeStruct((_NW, _ACC), jnp.float32)],
        mesh=mesh,
        compiler_params=pltpu.CompilerParams(needs_layout_passes=False),
        scratch_types=[
            pltpu.VMEM((_ACC,), jnp.float32),
            pltpu.VMEM((_SCH * 4,), jnp.float32),
            pltpu.VMEM((_SCH,), jnp.int32),
        ],
    )
    return gather, scatter


def _mlp_call(xs, xd, dist2, vech3, *weights):
    grid = (_E // _BE,)
    edge_spec = lambda width: pl.BlockSpec((_BE, width), lambda i: (i, 0))
    w_spec = lambda a, b: pl.BlockSpec((a, b), lambda i: (0, 0))
    in_specs = [
        edge_spec(_EMBED), edge_spec(_EMBED), edge_spec(1), edge_spec(3),
        w_spec(_EMBED, 2 * _HID), w_spec(_EMBED, 2 * _HID),
        w_spec(_NG, 2 * _HID), w_spec(1, 2 * _HID),
        w_spec(_HID, _HID), w_spec(1, _HID),
        w_spec(_HID, 1), w_spec(1, 1),
        w_spec(_HID, _HID), w_spec(1, _HID),
        w_spec(_HID, 1), w_spec(1, 1),
    ]
    return pl.pallas_call(
        _mlp_body,
        grid=grid,
        in_specs=in_specs,
        out_specs=pl.BlockSpec((_BE, 4), lambda i: (i, 0)),
        out_shape=jax.ShapeDtypeStruct((_E, 4), jnp.float32),
    )(xs, xd, dist2, vech3, *weights)


def _finish_call(x, batch2):
    return pl.pallas_call(
        _finish_body,
        in_specs=[pl.BlockSpec((_NW, _ACC), lambda: (0, 0)),
                  pl.BlockSpec((1, _NP), lambda: (0, 0))],
        out_specs=[pl.BlockSpec((_NGRAPH, 1), lambda: (0, 0)),
                   pl.BlockSpec((3, _NP), lambda: (0, 0))],
        out_shape=[jax.ShapeDtypeStruct((_NGRAPH, 1), jnp.float32),
                   jax.ShapeDtypeStruct((3, _NP), jnp.float32)],
    )(x, batch2)


def kernel(x, edge_index, batch, dist, vec_hat,
           W_rbf, b_rbf,
           e_Win, e_bin, e_Wh, e_bh, e_Wout, e_bout,
           f_Win, f_bin, f_Wh, f_bh, f_Wout, f_bout):
    gather_call, scatter_call = _sc_calls()
    src = edge_index[0]
    dst = edge_index[1]
    dist2 = dist.reshape(_E, 1)
    bf = jnp.bfloat16
    W1 = jnp.concatenate([e_Win[:_EMBED], f_Win[:_EMBED]], axis=1)
    W2 = jnp.concatenate(
        [e_Win[_EMBED:2 * _EMBED], f_Win[_EMBED:2 * _EMBED]], axis=1)
    W3 = jnp.concatenate([e_Win[2 * _EMBED:], f_Win[2 * _EMBED:]], axis=1)
    Wg = W_rbf @ W3
    b1 = (b_rbf @ W3 + jnp.concatenate([e_bin, f_bin])).reshape(1, 2 * _HID)
    weights = (
        W1.astype(bf), W2.astype(bf), Wg.astype(bf), b1,
        e_Wh.astype(bf), e_bh.reshape(1, _HID),
        e_Wout.astype(bf), e_bout.reshape(1, 1),
        f_Wh.astype(bf), f_bh.reshape(1, _HID),
        f_Wout.astype(bf), f_bout.reshape(1, 1))
    xs, xd = gather_call(x, src, dst)
    pairs = _mlp_call(xs, xd, dist2, vec_hat, *weights)
    (partials,) = scatter_call(pairs.reshape(_E * 4), src)
    batch_p = jnp.full((1, _NP), _NGRAPH, jnp.int32).at[0, :_N].set(batch)
    energy, ft = _finish_call(partials, batch_p)
    forces = ft[:, :_N].T
    return (energy, forces)


# speedup vs baseline: 1.2902x; 1.0003x over previous
import functools

import jax
import jax.numpy as jnp
from jax import lax
from jax.experimental import pallas as pl
from jax.experimental.pallas import tpu as pltpu
from jax.experimental.pallas import tpu_sc as plsc

_N = 10000
_E = 320000
_EMBED = 128
_HID = 256
_NG = 50
_RBF_R = 12.0
_AVG_LEN = 60.0
_CONN = 32.0
_NGRAPH = 64

_NC = 2
_NS = 16
_L = 16
_NW = _NC * _NS
_EPW = _E // _NW

_GCH = 80
_GITER = _EPW // _GCH

_SCH = 400
_SITER = _EPW // _SCH

_NP = 10240
_ACC = 4 * _NP

_BE = 2560


_XW = _EMBED // 2


def _gather_body(x_hbm, src_hbm, dst_hbm, xs_out, xd_out,
                 idx_s0, idx_d0,
                 rows_s0, rows_s1, rows_d0, rows_d1,
                 gs0, gs1, gd0, gd1, ws0, ws1, wd0, wd1):
    wid = lax.axis_index("s") * _NC + lax.axis_index("c")
    base0 = wid * _EPW
    rows_s = (rows_s0, rows_s1)
    rows_d = (rows_d0, rows_d1)
    gsem_s = (gs0, gs1)
    gsem_d = (gd0, gd1)
    wsem_s = (ws0, ws1)
    wsem_d = (wd0, wd1)

    pltpu.sync_copy(src_hbm.at[pl.ds(base0, _EPW)], idx_s0)
    pltpu.sync_copy(dst_hbm.at[pl.ds(base0, _EPW)], idx_d0)

    def start_gather(i, par):
        isl = pl.ds(i * _GCH, _GCH)
        pltpu.async_copy(x_hbm.at[idx_s0.at[isl]], rows_s[par], gsem_s[par])
        pltpu.async_copy(x_hbm.at[idx_d0.at[isl]], rows_d[par], gsem_d[par])

    def wait_gather(par):
        zsl = pl.ds(0, _GCH)
        pltpu.make_async_copy(
            x_hbm.at[idx_s0.at[zsl]], rows_s[par], gsem_s[par]).wait()
        pltpu.make_async_copy(
            x_hbm.at[idx_d0.at[zsl]], rows_d[par], gsem_d[par]).wait()

    def start_writeout(i, par):
        dst = pl.ds(base0 + i * _GCH, _GCH)
        pltpu.async_copy(rows_s[par], xs_out.at[dst], wsem_s[par])
        pltpu.async_copy(rows_d[par], xd_out.at[dst], wsem_d[par])

    def wait_writeout(par):
        dst = pl.ds(0, _GCH)
        pltpu.make_async_copy(rows_s[par], xs_out.at[dst], wsem_s[par]).wait()
        pltpu.make_async_copy(rows_d[par], xd_out.at[dst], wsem_d[par]).wait()

    def advance(i, par):
        nxt = 1 - par

        @pl.when(i + 1 < _GITER)
        def _():
            @pl.when(i >= 1)
            def _():
                wait_writeout(nxt)

            start_gather(i + 1, nxt)

        wait_gather(par)
        start_writeout(i, par)

    start_gather(0, 0)

    def step(g, carry):
        advance(2 * g, 0)
        advance(2 * g + 1, 1)
        return carry

    lax.fori_loop(0, _GITER // 2, step, 0)
    wait_gather(0)
    start_writeout(_GITER - 1, 0)
    wait_writeout(1)
    wait_writeout(0)


def _scatter_body(pairs_hbm, src_hbm, part_out, acc, pairs_v, src_v):
    wid = lax.axis_index("s") * _NC + lax.axis_index("c")
    zero = jnp.zeros((_L,), jnp.float32)

    def zstep(i, carry):
        acc[pl.ds(i * _L, _L)] = zero
        return carry

    lax.fori_loop(0, _ACC // _L, zstep, 0)

    lane = lax.iota(jnp.int32, _L)
    eo = lane >> 2
    fld = lane & 3

    def chunk(c, carry):
        base = wid * _EPW + c * _SCH
        pltpu.sync_copy(src_hbm.at[pl.ds(base, _SCH)], src_v)
        pltpu.sync_copy(pairs_hbm.at[pl.ds(base * 4, _SCH * 4)], pairs_v)

        def grp(j, icarry):
            vals = pairs_v[pl.ds(j * _L, _L)]
            s = plsc.load_gather(src_v, [j * 4 + eo])
            tgt = fld * _NP + s
            for e in range(4):
                plsc.addupdate_scatter(acc, [tgt], vals, mask=eo == e)
            return icarry

        lax.fori_loop(0, _SCH // 4, grp, 0)
        return carry

    lax.fori_loop(0, _SITER, chunk, 0)
    pltpu.sync_copy(acc, part_out.at[wid])


def _mlp_body(xs, xd, dist, vech, W1, W2, Wg, b1,
              e_Wh, e_bh, e_Wout, e_bout,
              f_Wh, f_bh, f_Wout, f_bout, out_ref):
    step = _RBF_R / (_NG - 1)
    offs = lax.broadcasted_iota(jnp.int32, (1, _NG), 1).astype(jnp.float32) * step
    coeff = -0.5 / step**2
    g = jnp.exp(coeff * (dist[...] - offs) ** 2)

    def silu(v):
        return 0.5 * v * (1.0 + jnp.tanh(0.5 * v))

    h = (jnp.dot(xs[...].astype(jnp.bfloat16), W1[...],
                 preferred_element_type=jnp.float32)
         + jnp.dot(xd[...].astype(jnp.bfloat16), W2[...],
                   preferred_element_type=jnp.float32)
         + jnp.dot(g.astype(jnp.bfloat16), Wg[...],
                   preferred_element_type=jnp.float32)
         + b1[...])
    h = silu(h)

    def head(h1, Wh, bh, Wout, bout):
        h2 = jnp.dot(h1.astype(jnp.bfloat16), Wh[...],
                     preferred_element_type=jnp.float32) + bh[...]
        h2 = h1 + silu(h2)
        return jnp.dot(h2.astype(jnp.bfloat16), Wout[...],
                       preferred_element_type=jnp.float32) + bout[...]

    ep = head(h[:, :_HID], e_Wh, e_bh, e_Wout, e_bout) * (1.0 / (_AVG_LEN * _CONN))
    fp = head(h[:, _HID:], f_Wh, f_bh, f_Wout, f_bout) * (1.0 / _CONN)
    out_ref[...] = jnp.concatenate([ep, fp * vech[...]], axis=1)


def _finish_body(x_ref, b_ref, energy_ref, ft_ref):
    xv = x_ref[...]
    e_node = jnp.sum(xv[:, 0:_NP], axis=0, keepdims=True)
    ft_ref[0:1, :] = jnp.sum(xv[:, _NP:2 * _NP], axis=0, keepdims=True)
    ft_ref[1:2, :] = jnp.sum(xv[:, 2 * _NP:3 * _NP], axis=0, keepdims=True)
    ft_ref[2:3, :] = jnp.sum(xv[:, 3 * _NP:4 * _NP], axis=0, keepdims=True)
    gid = lax.broadcasted_iota(jnp.int32, (_NGRAPH, 1), 0)
    onehot = (b_ref[...] == gid).astype(jnp.float32)
    energy_ref[...] = lax.dot_general(
        onehot, e_node, (((1,), (1,)), ((), ())),
        preferred_element_type=jnp.float32)


@functools.lru_cache(maxsize=None)
def _sc_calls():
    mesh = plsc.VectorSubcoreMesh(core_axis_name="c", subcore_axis_name="s")
    gather = pl.kernel(
        _gather_body,
        out_type=[jax.ShapeDtypeStruct((_E, _EMBED), jnp.float32),
                  jax.ShapeDtypeStruct((_E, _EMBED), jnp.float32)],
        mesh=mesh,
        scratch_types=(
            [pltpu.VMEM((_EPW,), jnp.int32)] * 2
            + [pltpu.VMEM((_GCH, _EMBED), jnp.float32)] * 4
            + [pltpu.SemaphoreType.DMA] * 8
        ),
    )
    scatter = pl.kernel(
        _scatter_body,
        out_type=[jax.ShapeDtypeStruct((_NW, _ACC), jnp.float32)],
        mesh=mesh,
        compiler_params=pltpu.CompilerParams(needs_layout_passes=False),
        scratch_types=[
            pltpu.VMEM((_ACC,), jnp.float32),
            pltpu.VMEM((_SCH * 4,), jnp.float32),
            pltpu.VMEM((_SCH,), jnp.int32),
        ],
    )
    return gather, scatter


def _mlp_call(xs, xd, dist2, vech3, *weights):
    grid = (_E // _BE,)
    edge_spec = lambda width: pl.BlockSpec((_BE, width), lambda i: (i, 0))
    w_spec = lambda a, b: pl.BlockSpec((a, b), lambda i: (0, 0))
    in_specs = [
        edge_spec(_EMBED), edge_spec(_EMBED), edge_spec(1), edge_spec(3),
        w_spec(_EMBED, 2 * _HID), w_spec(_EMBED, 2 * _HID),
        w_spec(_NG, 2 * _HID), w_spec(1, 2 * _HID),
        w_spec(_HID, _HID), w_spec(1, _HID),
        w_spec(_HID, 1), w_spec(1, 1),
        w_spec(_HID, _HID), w_spec(1, _HID),
        w_spec(_HID, 1), w_spec(1, 1),
    ]
    return pl.pallas_call(
        _mlp_body,
        grid=grid,
        in_specs=in_specs,
        out_specs=pl.BlockSpec((_BE, 4), lambda i: (i, 0)),
        out_shape=jax.ShapeDtypeStruct((_E, 4), jnp.float32),
    )(xs, xd, dist2, vech3, *weights)


def _finish_call(x, batch2):
    return pl.pallas_call(
        _finish_body,
        in_specs=[pl.BlockSpec((_NW, _ACC), lambda: (0, 0)),
                  pl.BlockSpec((1, _NP), lambda: (0, 0))],
        out_specs=[pl.BlockSpec((_NGRAPH, 1), lambda: (0, 0)),
                   pl.BlockSpec((3, _NP), lambda: (0, 0))],
        out_shape=[jax.ShapeDtypeStruct((_NGRAPH, 1), jnp.float32),
                   jax.ShapeDtypeStruct((3, _NP), jnp.float32)],
    )(x, batch2)


def kernel(x, edge_index, batch, dist, vec_hat,
           W_rbf, b_rbf,
           e_Win, e_bin, e_Wh, e_bh, e_Wout, e_bout,
           f_Win, f_bin, f_Wh, f_bh, f_Wout, f_bout):
    gather_call, scatter_call = _sc_calls()
    src = edge_index[0]
    dst = edge_index[1]
    dist2 = dist.reshape(_E, 1)
    bf = jnp.bfloat16
    W1 = jnp.concatenate([e_Win[:_EMBED], f_Win[:_EMBED]], axis=1)
    W2 = jnp.concatenate(
        [e_Win[_EMBED:2 * _EMBED], f_Win[_EMBED:2 * _EMBED]], axis=1)
    W3 = jnp.concatenate([e_Win[2 * _EMBED:], f_Win[2 * _EMBED:]], axis=1)
    Wg = W_rbf @ W3
    b1 = (b_rbf @ W3 + jnp.concatenate([e_bin, f_bin])).reshape(1, 2 * _HID)
    weights = (
        W1.astype(bf), W2.astype(bf), Wg.astype(bf), b1,
        e_Wh.astype(bf), e_bh.reshape(1, _HID),
        e_Wout.astype(bf), e_bout.reshape(1, 1),
        f_Wh.astype(bf), f_bh.reshape(1, _HID),
        f_Wout.astype(bf), f_bout.reshape(1, 1))
    xs, xd = gather_call(x, src, dst)
    pairs = _mlp_call(xs, xd, dist2, vec_hat, *weights)
    (partials,) = scatter_call(pairs.reshape(_E * 4), src)
    batch_p = jnp.full((1, _NP), _NGRAPH, jnp.int32).at[0, :_N].set(batch)
    energy, ft = _finish_call(partials, batch_p)
    forces = ft[:, :_N].T
    return (energy, forces)
